# Initial kernel scaffold; baseline (speedup 1.0000x reference)
import jax
import jax.numpy as jnp
from jax.experimental import pallas as pl


def kernel(input, target):
    raise NotImplementedError("write your pallas kernel here")


# speedup vs baseline: 5.7990x; 5.7990x over previous
import jax
import jax.numpy as jnp
from jax import lax
from jax.experimental import pallas as pl
from jax.experimental.pallas import tpu as pltpu
from jax.experimental.pallas import tpu_sc as plsc

B, L = 1024, 256
NC, NS, LANES = 2, 16, 16
NW = NC * NS
ROWS = B // NW
GROUPS = ROWS // LANES


def _bpmll_body(x_hbm, t_hbm, out_hbm, x_v, t_v, o_v):
    wid = lax.axis_index("s") * NC + lax.axis_index("c")
    base = wid * (ROWS * L)
    pltpu.sync_copy(x_hbm.at[pl.ds(base, ROWS * L)], x_v)
    pltpu.sync_copy(t_hbm.at[pl.ds(base, ROWS * L)], t_v)

    lanes = jnp.arange(LANES, dtype=jnp.int32)

    def group_body(g, _):
        row_base = (g * LANES + lanes) * L

        def col_body(c, carry):
            en, ep, npos = carry
            idx = row_base + c
            xv = plsc.load_gather(x_v, [idx])
            tv = plsc.load_gather(t_v, [idx])
            pos = tv == 1.0
            en = en + jnp.where(pos, 0.0, jnp.exp(xv))
            ep = ep + jnp.where(pos, jnp.exp(-xv), 0.0)
            return en, ep, npos + tv

        z = jnp.zeros((LANES,), jnp.float32)
        en, ep, npos = lax.fori_loop(0, L, col_body, (z, z, z))
        loss_vec = en * ep / (npos * (float(L) - npos))
        o_v[pl.ds(g * LANES, LANES)] = loss_vec
        return 0

    lax.fori_loop(0, GROUPS, group_body, 0)
    pltpu.sync_copy(o_v, out_hbm.at[pl.ds(wid * ROWS, ROWS)])


_sc_fn = None


def _get_sc_fn():
    global _sc_fn
    if _sc_fn is None:
        mesh = plsc.VectorSubcoreMesh(
            core_axis_name="c", subcore_axis_name="s", num_cores=NC, num_subcores=NS
        )
        _sc_fn = pl.kernel(
            _bpmll_body,
            out_type=jax.ShapeDtypeStruct((B,), jnp.float32),
            mesh=mesh,
            scratch_types=[
                pltpu.VMEM((ROWS * L,), jnp.float32),
                pltpu.VMEM((ROWS * L,), jnp.float32),
                pltpu.VMEM((ROWS,), jnp.float32),
            ],
            compiler_params=pltpu.CompilerParams(needs_layout_passes=False),
        )
    return _sc_fn


def kernel(input, target):
    x = input.reshape(B * L)
    t = target.astype(jnp.float32).reshape(B * L)
    losses = _get_sc_fn()(x, t)
    return jnp.sum(losses)
